# baseline (speedup 1.0000x reference)
import functools

import jax
import jax.numpy as jnp
from jax import lax
from jax.experimental import pallas as pl
from jax.experimental.pallas import tpu as pltpu
from jax.experimental.pallas import tpu_sc as plsc

NA = 10000
NE = 320000
F = 128
G = 50
CUTOFF = 5.0
LOG2 = 0.6931471805599453
GWIDTH = CUTOFF / (G - 1)
GCOEFF = -0.5 / (GWIDTH * GWIDTH)

KTAB = 8192
DELTA = CUTOFF / KTAB
INV_DELTA = KTAB / CUTOFF
TROWS = KTAB + 8

NC = 2
NS = 16
NW = NC * NS
EPW = NE // NW
CH = 80
NCHUNK = EPW // CH
RPT = 624
TAIL = NA - RPT * NS


def _pre_body(r_ref, wa_ref, wd2_ref, bd2_ref, rf_ref, tab_ref):
    rf_ref[...] = jnp.dot(r_ref[...], wa_ref[...],
                          preferred_element_type=jnp.float32)
    dist = lax.broadcasted_iota(jnp.int32, (TROWS, 64), 0).astype(jnp.float32) * DELTA
    gpos = lax.broadcasted_iota(jnp.int32, (TROWS, 64), 1).astype(jnp.float32) * GWIDTH
    eexp = jnp.exp(GCOEFF * (dist - gpos) ** 2)
    tab_ref[...] = jnp.dot(eexp, wd2_ref[...],
                           preferred_element_type=jnp.float32) + bd2_ref[...]


_tc_pre = pl.pallas_call(
    _pre_body,
    out_shape=[
        jax.ShapeDtypeStruct((NA, F), jnp.float32),
        jax.ShapeDtypeStruct((TROWS, F), jnp.float32),
    ],
)


def _post_body(p0_ref, p1_ref, w1_ref, b1_ref, w2_ref, b2_ref, o_ref):
    y = p0_ref[...] + p1_ref[...]
    h = jnp.dot(y, w1_ref[...], preferred_element_type=jnp.float32) + b1_ref[...]
    h = jnp.maximum(h, 0.0) + jnp.log(1.0 + jnp.exp(-jnp.abs(h))) - LOG2
    o_ref[...] = jnp.dot(h, w2_ref[...],
                         preferred_element_type=jnp.float32) + b2_ref[...]


_tc_post = pl.pallas_call(
    _post_body,
    out_shape=jax.ShapeDtypeStruct((NA, F), jnp.float32),
)


@functools.cache
def _build_sc_main():
  mesh = plsc.VectorSubcoreMesh(core_axis_name="c", subcore_axis_name="s",
                                num_cores=NC, num_subcores=NS)

  @functools.partial(
      pl.kernel,
      out_type=jax.ShapeDtypeStruct((NC, NA, F), jnp.float32),
      mesh=mesh,
      scratch_types=[
          pltpu.VMEM((2, CH), jnp.int32),
          pltpu.VMEM((2, CH), jnp.int32),
          pltpu.VMEM((2, CH), jnp.float32),
          pltpu.VMEM((2, CH), jnp.int32),
          pltpu.VMEM((2, CH, F), jnp.float32),
          pltpu.VMEM((2, CH, F), jnp.float32),
          pltpu.VMEM_SHARED((NA, F), jnp.float32),
          pltpu.SemaphoreType.DMA,
          pltpu.SemaphoreType.DMA,
          pltpu.SemaphoreType.DMA,
          pltpu.SemaphoreType.DMA,
      ],
  )
  def _sc_main(src_hbm, dst_hbm, d_hbm, tab_hbm, rf_hbm, zero_hbm, out_hbm,
               src_v, dst_v, d_v, k_v, rows_v, trows_v, acc_sh,
               sem_i0, sem_i1, sem_g0, sem_g1):
    c = lax.axis_index("c")
    s = lax.axis_index("s")
    wid = s * NC + c
    sems_i = (sem_i0, sem_i1)
    sems_g = (sem_g0, sem_g1)

    pltpu.sync_copy(zero_hbm.at[pl.ds(s * RPT, RPT)],
                    acc_sh.at[pl.ds(s * RPT, RPT)])

    @pl.when(s == NS - 1)
    def _zero_tail():
      pltpu.sync_copy(zero_hbm.at[pl.ds(RPT * NS, TAIL)],
                      acc_sh.at[pl.ds(RPT * NS, TAIL)])

    base = wid * EPW

    def idx_load(i, b):
      off = base + i * CH
      pltpu.async_copy(src_hbm.at[pl.ds(off, CH)], src_v.at[b], sems_i[b])
      pltpu.async_copy(dst_hbm.at[pl.ds(off, CH)], dst_v.at[b], sems_i[b])
      pltpu.async_copy(d_hbm.at[pl.ds(off, CH)], d_v.at[b], sems_i[b])

    def drain_idx(b):
      pltpu.make_async_copy(src_hbm.at[pl.ds(0, CH)], src_v.at[b],
                            sems_i[b]).wait()
      pltpu.make_async_copy(src_hbm.at[pl.ds(0, CH)], dst_v.at[b],
                            sems_i[b]).wait()
      pltpu.make_async_copy(d_hbm.at[pl.ds(0, CH)], d_v.at[b],
                            sems_i[b]).wait()

    def kcomp(b):
      for j in range(CH // 16):
        sl = pl.ds(j * 16, 16)
        t = d_v[b, sl] * INV_DELTA + 0.5
        k_v[b, sl] = jnp.minimum(t.astype(jnp.int32), TROWS - 1)

    def fire(i, b):
      del i
      pltpu.async_copy(rf_hbm.at[src_v.at[b]], rows_v.at[b], sems_g[b])
      pltpu.async_copy(tab_hbm.at[k_v.at[b]], trows_v.at[b], sems_g[b])

    def drain_gather(b):
      pltpu.make_async_copy(rf_hbm.at[src_v.at[b]], rows_v.at[b],
                            sems_g[b]).wait()
      pltpu.make_async_copy(rf_hbm.at[src_v.at[b]], trows_v.at[b],
                            sems_g[b]).wait()

    def mul(b):
      @pl.loop(0, CH, unroll=2)
      def _mul(e2):
        for l in range(F // 16):
          sl = pl.ds(l * 16, 16)
          rows_v[b, e2, sl] = rows_v[b, e2, sl] * trows_v[b, e2, sl]

    def scatter(i, b):
      del i
      pltpu.sync_copy(rows_v.at[b], acc_sh.at[dst_v.at[b]], add=True)

    plsc.subcore_barrier()

    pltpu.sync_copy(src_hbm.at[pl.ds(base, CH)], src_v.at[0])
    pltpu.sync_copy(dst_hbm.at[pl.ds(base, CH)], dst_v.at[0])
    pltpu.sync_copy(d_hbm.at[pl.ds(base, CH)], d_v.at[0])
    kcomp(0)
    fire(0, 0)
    idx_load(1, 1)

    @pl.loop(0, NCHUNK - 1, step=2)
    def _chunk(i):
      drain_idx(1)
      kcomp(1)
      fire(i + 1, 1)
      drain_gather(0)
      mul(0)
      scatter(i, 0)

      @pl.when(i + 2 < NCHUNK)
      def _load0():
        idx_load(i + 2, 0)

      drain_gather(1)
      mul(1)
      scatter(i + 1, 1)

      @pl.when(i + 2 < NCHUNK)
      def _fire0():
        drain_idx(0)
        kcomp(0)
        fire(i + 2, 0)

      @pl.when(i + 3 < NCHUNK)
      def _load1():
        idx_load(i + 3, 1)

    drain_gather(0)
    mul(0)
    scatter(NCHUNK - 1, 0)

    plsc.subcore_barrier()
    pltpu.sync_copy(acc_sh.at[pl.ds(s * RPT, RPT)],
                    out_hbm.at[c, pl.ds(s * RPT, RPT)])

    @pl.when(s == NS - 1)
    def _out_tail():
      pltpu.sync_copy(acc_sh.at[pl.ds(RPT * NS, TAIL)],
                      out_hbm.at[c, pl.ds(RPT * NS, TAIL)])

  return _sc_main


def kernel(r, e, a, Wd1, bd1, Wd2, bd2, Wa, W1, b1, W2, b2):
    del Wd1, bd1
    a = a.astype(jnp.int32)
    src = a[:, 1]
    dst = a[:, 0]
    d = e[:, 0]
    wd2p = jnp.zeros((64, F), jnp.float32).at[:G].set(Wd2)
    rf, tab = _tc_pre(r, Wa, wd2p, bd2.reshape(1, F))
    zeros = jnp.zeros((NA, F), jnp.float32)
    part = _build_sc_main()(src, dst, d, tab, rf, zeros)
    return _tc_post(part[0], part[1], W1, b1.reshape(1, F),
                    W2, b2.reshape(1, F))


# speedup vs baseline: 1.0002x; 1.0002x over previous
import functools

import jax
import jax.numpy as jnp
from jax import lax
from jax.experimental import pallas as pl
from jax.experimental.pallas import tpu as pltpu
from jax.experimental.pallas import tpu_sc as plsc

NA = 10000
NE = 320000
F = 128
G = 50
CUTOFF = 5.0
LOG2 = 0.6931471805599453
GWIDTH = CUTOFF / (G - 1)
GCOEFF = -0.5 / (GWIDTH * GWIDTH)

KTAB = 8192
DELTA = CUTOFF / KTAB
INV_DELTA = KTAB / CUTOFF
TROWS = KTAB + 8

NC = 2
NS = 16
NW = NC * NS
EPW = NE // NW
CH = 80
NCHUNK = EPW // CH
RPT = 624
TAIL = NA - RPT * NS


def _pre_body(r_ref, wa_ref, wd2_ref, bd2_ref, rf_ref, tab_ref):
    rf_ref[...] = jnp.dot(r_ref[...], wa_ref[...],
                          preferred_element_type=jnp.float32)
    dist = lax.broadcasted_iota(jnp.int32, (TROWS, 64), 0).astype(jnp.float32) * DELTA
    gpos = lax.broadcasted_iota(jnp.int32, (TROWS, 64), 1).astype(jnp.float32) * GWIDTH
    eexp = jnp.exp(GCOEFF * (dist - gpos) ** 2)
    tab_ref[...] = jnp.dot(eexp, wd2_ref[...],
                           preferred_element_type=jnp.float32) + bd2_ref[...]


_tc_pre = pl.pallas_call(
    _pre_body,
    out_shape=[
        jax.ShapeDtypeStruct((NA, F), jnp.float32),
        jax.ShapeDtypeStruct((TROWS, F), jnp.float32),
    ],
)


def _post_body(p0_ref, p1_ref, w1_ref, b1_ref, w2_ref, b2_ref, o_ref):
    y = p0_ref[...] + p1_ref[...]
    h = jnp.dot(y, w1_ref[...], preferred_element_type=jnp.float32) + b1_ref[...]
    h = jnp.maximum(h, 0.0) + jnp.log(1.0 + jnp.exp(-jnp.abs(h))) - LOG2
    o_ref[...] = jnp.dot(h, w2_ref[...],
                         preferred_element_type=jnp.float32) + b2_ref[...]


_tc_post = pl.pallas_call(
    _post_body,
    out_shape=jax.ShapeDtypeStruct((NA, F), jnp.float32),
)


@functools.cache
def _build_sc_main():
  mesh = plsc.VectorSubcoreMesh(core_axis_name="c", subcore_axis_name="s",
                                num_cores=NC, num_subcores=NS)

  @functools.partial(
      pl.kernel,
      out_type=jax.ShapeDtypeStruct((NC, NA, F), jnp.float32),
      mesh=mesh,
      scratch_types=[
          pltpu.VMEM((2, CH), jnp.int32),
          pltpu.VMEM((2, CH), jnp.int32),
          pltpu.VMEM((2, CH), jnp.float32),
          pltpu.VMEM((2, CH), jnp.int32),
          pltpu.VMEM((2, CH, F), jnp.float32),
          pltpu.VMEM((2, CH, F), jnp.float32),
          pltpu.VMEM_SHARED((NA, F), jnp.float32),
          pltpu.SemaphoreType.DMA,
          pltpu.SemaphoreType.DMA,
          pltpu.SemaphoreType.DMA,
          pltpu.SemaphoreType.DMA,
      ],
  )
  def _sc_main(src_hbm, dst_hbm, d_hbm, tab_hbm, rf_hbm, zero_hbm, out_hbm,
               src_v, dst_v, d_v, k_v, rows_v, trows_v, acc_sh,
               sem_i0, sem_i1, sem_g0, sem_g1):
    c = lax.axis_index("c")
    s = lax.axis_index("s")
    wid = s * NC + c
    sems_i = (sem_i0, sem_i1)
    sems_g = (sem_g0, sem_g1)

    pltpu.sync_copy(zero_hbm.at[pl.ds(s * RPT, RPT)],
                    acc_sh.at[pl.ds(s * RPT, RPT)])

    @pl.when(s == NS - 1)
    def _zero_tail():
      pltpu.sync_copy(zero_hbm.at[pl.ds(RPT * NS, TAIL)],
                      acc_sh.at[pl.ds(RPT * NS, TAIL)])

    base = wid * EPW

    def idx_load(i, b):
      off = base + i * CH
      pltpu.async_copy(src_hbm.at[pl.ds(off, CH)], src_v.at[b], sems_i[b])
      pltpu.async_copy(dst_hbm.at[pl.ds(off, CH)], dst_v.at[b], sems_i[b])
      pltpu.async_copy(d_hbm.at[pl.ds(off, CH)], d_v.at[b], sems_i[b])

    def drain_idx(b):
      pltpu.make_async_copy(src_hbm.at[pl.ds(0, CH)], src_v.at[b],
                            sems_i[b]).wait()
      pltpu.make_async_copy(src_hbm.at[pl.ds(0, CH)], dst_v.at[b],
                            sems_i[b]).wait()
      pltpu.make_async_copy(d_hbm.at[pl.ds(0, CH)], d_v.at[b],
                            sems_i[b]).wait()

    def kcomp(b):
      for j in range(CH // 16):
        sl = pl.ds(j * 16, 16)
        t = d_v[b, sl] * INV_DELTA + 0.5
        k_v[b, sl] = jnp.minimum(t.astype(jnp.int32), TROWS - 1)

    def fire(i, b):
      del i
      pltpu.async_copy(rf_hbm.at[src_v.at[b]], rows_v.at[b], sems_g[b])
      pltpu.async_copy(tab_hbm.at[k_v.at[b]], trows_v.at[b], sems_g[b])

    def drain_gather(b):
      pltpu.make_async_copy(zero_hbm.at[pl.ds(0, CH)], rows_v.at[b],
                            sems_g[b]).wait()
      pltpu.make_async_copy(zero_hbm.at[pl.ds(0, CH)], trows_v.at[b],
                            sems_g[b]).wait()

    def mul(b):
      @pl.loop(0, CH, unroll=2)
      def _mul(e2):
        for l in range(F // 16):
          sl = pl.ds(l * 16, 16)
          rows_v[b, e2, sl] = rows_v[b, e2, sl] * trows_v[b, e2, sl]

    def scatter(i, b):
      del i
      pltpu.sync_copy(rows_v.at[b], acc_sh.at[dst_v.at[b]], add=True)

    plsc.subcore_barrier()

    pltpu.sync_copy(src_hbm.at[pl.ds(base, CH)], src_v.at[0])
    pltpu.sync_copy(dst_hbm.at[pl.ds(base, CH)], dst_v.at[0])
    pltpu.sync_copy(d_hbm.at[pl.ds(base, CH)], d_v.at[0])
    kcomp(0)
    fire(0, 0)
    idx_load(1, 1)

    @pl.loop(0, NCHUNK - 1, step=2)
    def _chunk(i):
      drain_idx(1)
      kcomp(1)
      fire(i + 1, 1)
      drain_gather(0)
      mul(0)
      scatter(i, 0)

      @pl.when(i + 2 < NCHUNK)
      def _load0():
        idx_load(i + 2, 0)

      drain_gather(1)
      mul(1)
      scatter(i + 1, 1)

      @pl.when(i + 2 < NCHUNK)
      def _fire0():
        drain_idx(0)
        kcomp(0)
        fire(i + 2, 0)

      @pl.when(i + 3 < NCHUNK)
      def _load1():
        idx_load(i + 3, 1)

    drain_gather(0)
    mul(0)
    scatter(NCHUNK - 1, 0)

    plsc.subcore_barrier()
    pltpu.sync_copy(acc_sh.at[pl.ds(s * RPT, RPT)],
                    out_hbm.at[c, pl.ds(s * RPT, RPT)])

    @pl.when(s == NS - 1)
    def _out_tail():
      pltpu.sync_copy(acc_sh.at[pl.ds(RPT * NS, TAIL)],
                      out_hbm.at[c, pl.ds(RPT * NS, TAIL)])

  return _sc_main


def kernel(r, e, a, Wd1, bd1, Wd2, bd2, Wa, W1, b1, W2, b2):
    del Wd1, bd1
    a = a.astype(jnp.int32)
    src = a[:, 1]
    dst = a[:, 0]
    d = e[:, 0]
    wd2p = jnp.zeros((64, F), jnp.float32).at[:G].set(Wd2)
    rf, tab = _tc_pre(r, Wa, wd2p, bd2.reshape(1, F))
    zeros = jnp.zeros((NA, F), jnp.float32)
    part = _build_sc_main()(src, dst, d, tab, rf, zeros)
    return _tc_post(part[0], part[1], W1, b1.reshape(1, F),
                    W2, b2.reshape(1, F))


# speedup vs baseline: 1.2914x; 1.2912x over previous
import functools

import jax
import jax.numpy as jnp
from jax import lax
from jax.experimental import pallas as pl
from jax.experimental.pallas import tpu as pltpu
from jax.experimental.pallas import tpu_sc as plsc

NA = 10000
NE = 320000
F = 128
G = 50
CUTOFF = 5.0
LOG2 = 0.6931471805599453
GWIDTH = CUTOFF / (G - 1)
GCOEFF = -0.5 / (GWIDTH * GWIDTH)

KTAB = 8192
DELTA = CUTOFF / KTAB
INV_DELTA = KTAB / CUTOFF
TROWS = KTAB + 8

NC = 2
NS = 16
NW = NC * NS
EPW = NE // NW
CH = 80
NCHUNK = EPW // CH
RPT = 624
TAIL = NA - RPT * NS


def _pre_body(r_ref, wa_ref, wd2_ref, bd2_ref, rf_ref, tab_ref):
    rf_ref[...] = jnp.dot(r_ref[...], wa_ref[...],
                          preferred_element_type=jnp.float32)
    dist = lax.broadcasted_iota(jnp.int32, (TROWS, 64), 0).astype(jnp.float32) * DELTA
    gpos = lax.broadcasted_iota(jnp.int32, (TROWS, 64), 1).astype(jnp.float32) * GWIDTH
    eexp = jnp.exp(GCOEFF * (dist - gpos) ** 2)
    tab_ref[...] = jnp.dot(eexp, wd2_ref[...],
                           preferred_element_type=jnp.float32) + bd2_ref[...]


_tc_pre = pl.pallas_call(
    _pre_body,
    out_shape=[
        jax.ShapeDtypeStruct((NA, F), jnp.float32),
        jax.ShapeDtypeStruct((TROWS, F), jnp.float32),
    ],
)


def _post_body(p0_ref, p1_ref, w1_ref, b1_ref, w2_ref, b2_ref, o_ref):
    y = p0_ref[...] + p1_ref[...]
    h = jnp.dot(y, w1_ref[...], preferred_element_type=jnp.float32) + b1_ref[...]
    h = jnp.maximum(h, 0.0) + jnp.log(1.0 + jnp.exp(-jnp.abs(h))) - LOG2
    o_ref[...] = jnp.dot(h, w2_ref[...],
                         preferred_element_type=jnp.float32) + b2_ref[...]


_tc_post = pl.pallas_call(
    _post_body,
    out_shape=jax.ShapeDtypeStruct((NA, F), jnp.float32),
)


@functools.cache
def _build_sc_main():
  mesh = plsc.VectorSubcoreMesh(core_axis_name="c", subcore_axis_name="s",
                                num_cores=NC, num_subcores=NS)

  @functools.partial(
      pl.kernel,
      out_type=jax.ShapeDtypeStruct((NC, NA, F), jnp.float32),
      mesh=mesh,
      scratch_types=[
          pltpu.VMEM((3, CH), jnp.int32),
          pltpu.VMEM((CH,), jnp.int32),
          pltpu.VMEM((CH, F), jnp.float32),
          pltpu.VMEM((CH, F), jnp.float32),
          pltpu.VMEM_SHARED((NA, F), jnp.float32),
          pltpu.SemaphoreType.DMA,
          pltpu.SemaphoreType.DMA,
      ],
  )
  def _sc_main(idx3_hbm, tab_hbm, rf_hbm, zero_hbm, out_hbm,
               idx3_v, k_v, rows_v, trows_v, acc_sh, sem1, sem2):
    c = lax.axis_index("c")
    s = lax.axis_index("s")
    wid = s * NC + c

    pltpu.sync_copy(zero_hbm.at[pl.ds(s * RPT, RPT)],
                    acc_sh.at[pl.ds(s * RPT, RPT)])

    @pl.when(s == NS - 1)
    def _zero_tail():
      pltpu.sync_copy(zero_hbm.at[pl.ds(RPT * NS, TAIL)],
                      acc_sh.at[pl.ds(RPT * NS, TAIL)])

    plsc.subcore_barrier()

    cbase = wid * NCHUNK

    @pl.loop(0, NCHUNK)
    def _chunk(i):
      pltpu.sync_copy(idx3_hbm.at[cbase + i], idx3_v)

      for j in range(CH // 16):
        sl = pl.ds(j * 16, 16)
        t = lax.bitcast_convert_type(idx3_v[2, sl],
                                     jnp.float32) * INV_DELTA + 0.5
        k_v[sl] = jnp.minimum(t.astype(jnp.int32), TROWS - 1)

      cp1 = pltpu.async_copy(rf_hbm.at[idx3_v.at[0]], rows_v, sem1)
      cp2 = pltpu.async_copy(tab_hbm.at[k_v], trows_v, sem2)
      cp1.wait()
      cp2.wait()

      @pl.loop(0, CH)
      def _mul(e2):
        for l in range(F // 16):
          sl = pl.ds(l * 16, 16)
          rows_v[e2, sl] = rows_v[e2, sl] * trows_v[e2, sl]

      pltpu.sync_copy(rows_v, acc_sh.at[idx3_v.at[1]], add=True)

    plsc.subcore_barrier()
    pltpu.sync_copy(acc_sh.at[pl.ds(s * RPT, RPT)],
                    out_hbm.at[c, pl.ds(s * RPT, RPT)])

    @pl.when(s == NS - 1)
    def _out_tail():
      pltpu.sync_copy(acc_sh.at[pl.ds(RPT * NS, TAIL)],
                      out_hbm.at[c, pl.ds(RPT * NS, TAIL)])

  return _sc_main


def kernel(r, e, a, Wd1, bd1, Wd2, bd2, Wa, W1, b1, W2, b2):
    del Wd1, bd1
    a = a.astype(jnp.int32)
    dbits = lax.bitcast_convert_type(e[:, 0], jnp.int32)
    idx3 = jnp.stack([a[:, 1].reshape(-1, CH), a[:, 0].reshape(-1, CH),
                      dbits.reshape(-1, CH)], axis=1)
    wd2p = jnp.zeros((64, F), jnp.float32).at[:G].set(Wd2)
    rf, tab = _tc_pre(r, Wa, wd2p, bd2.reshape(1, F))
    zeros = jnp.zeros((NA, F), jnp.float32)
    part = _build_sc_main()(idx3, tab, rf, zeros)
    return _tc_post(part[0], part[1], W1, b1.reshape(1, F),
                    W2, b2.reshape(1, F))


# speedup vs baseline: 1.5779x; 1.2219x over previous
import functools

import jax
import jax.numpy as jnp
from jax import lax
from jax.experimental import pallas as pl
from jax.experimental.pallas import tpu as pltpu
from jax.experimental.pallas import tpu_sc as plsc

NA = 10000
NE = 320000
F = 128
G = 50
CUTOFF = 5.0
LOG2 = 0.6931471805599453
GWIDTH = CUTOFF / (G - 1)
GCOEFF = -0.5 / (GWIDTH * GWIDTH)

KTAB = 8192
DELTA = CUTOFF / KTAB
INV_DELTA = KTAB / CUTOFF
TROWS = KTAB + 8

NC = 2
NS = 16
NW = NC * NS
EPW = NE // NW
CH = 80
NCHUNK = EPW // CH
RPT = 624
TAIL = NA - RPT * NS


def _pre_body(r_ref, wa_ref, wd2_ref, bd2_ref, rf_ref, tab_ref):
    rf_ref[...] = jnp.dot(r_ref[...], wa_ref[...],
                          preferred_element_type=jnp.float32)
    dist = lax.broadcasted_iota(jnp.int32, (TROWS, 64), 0).astype(jnp.float32) * DELTA
    gpos = lax.broadcasted_iota(jnp.int32, (TROWS, 64), 1).astype(jnp.float32) * GWIDTH
    eexp = jnp.exp(GCOEFF * (dist - gpos) ** 2)
    tab_ref[...] = jnp.dot(eexp, wd2_ref[...],
                           preferred_element_type=jnp.float32) + bd2_ref[...]


_tc_pre = pl.pallas_call(
    _pre_body,
    out_shape=[
        jax.ShapeDtypeStruct((NA, F), jnp.float32),
        jax.ShapeDtypeStruct((TROWS, F), jnp.float32),
    ],
)


def _post_body(p0_ref, p1_ref, w1_ref, b1_ref, w2_ref, b2_ref, o_ref):
    y = p0_ref[...] + p1_ref[...]
    h = jnp.dot(y, w1_ref[...], preferred_element_type=jnp.float32) + b1_ref[...]
    h = jnp.maximum(h, 0.0) + jnp.log(1.0 + jnp.exp(-jnp.abs(h))) - LOG2
    o_ref[...] = jnp.dot(h, w2_ref[...],
                         preferred_element_type=jnp.float32) + b2_ref[...]


_tc_post = pl.pallas_call(
    _post_body,
    out_shape=jax.ShapeDtypeStruct((NA, F), jnp.float32),
)


@functools.cache
def _build_sc_main():
  mesh = plsc.VectorSubcoreMesh(core_axis_name="c", subcore_axis_name="s",
                                num_cores=NC, num_subcores=NS)

  @functools.partial(
      pl.kernel,
      out_type=jax.ShapeDtypeStruct((NC, NA, F), jnp.float32),
      mesh=mesh,
      scratch_types=[
          pltpu.VMEM((2, 3, CH), jnp.int32),
          pltpu.VMEM((2, CH), jnp.int32),
          pltpu.VMEM((2, CH, F), jnp.float32),
          pltpu.VMEM((2, CH, F), jnp.float32),
          pltpu.VMEM_SHARED((NA, F), jnp.float32),
          pltpu.SemaphoreType.DMA,
          pltpu.SemaphoreType.DMA,
      ],
  )
  def _sc_main(idx3_hbm, tab_hbm, rf_hbm, zero_hbm, out_hbm,
               idx3_v, k_v, rows_v, trows_v, acc_sh, sem1, sem2):
    c = lax.axis_index("c")
    s = lax.axis_index("s")
    wid = s * NC + c
    sems = (sem1, sem2)

    pltpu.sync_copy(zero_hbm.at[pl.ds(s * RPT, RPT)],
                    acc_sh.at[pl.ds(s * RPT, RPT)])

    @pl.when(s == NS - 1)
    def _zero_tail():
      pltpu.sync_copy(zero_hbm.at[pl.ds(RPT * NS, TAIL)],
                      acc_sh.at[pl.ds(RPT * NS, TAIL)])

    plsc.subcore_barrier()

    cbase = wid * NCHUNK

    def stage(i, b):
      pltpu.sync_copy(idx3_hbm.at[cbase + i], idx3_v.at[b])
      for j in range(CH // 16):
        sl = pl.ds(j * 16, 16)
        t = lax.bitcast_convert_type(idx3_v[b, 2, sl],
                                     jnp.float32) * INV_DELTA + 0.5
        k_v[b, sl] = jnp.minimum(t.astype(jnp.int32), TROWS - 1)
      cp1 = pltpu.async_copy(rf_hbm.at[idx3_v.at[b].at[0]],
                             rows_v.at[b], sems[b])
      cp2 = pltpu.async_copy(tab_hbm.at[k_v.at[b]], trows_v.at[b], sems[b])
      return cp1, cp2

    def finish(b, cp1, cp2):
      cp1.wait()
      cp2.wait()

      @pl.loop(0, CH)
      def _mul(e2):
        for l in range(F // 16):
          sl = pl.ds(l * 16, 16)
          rows_v[b, e2, sl] = rows_v[b, e2, sl] * trows_v[b, e2, sl]

      pltpu.sync_copy(rows_v.at[b], acc_sh.at[idx3_v.at[b].at[1]], add=True)

    @pl.loop(0, NCHUNK - 1, step=2)
    def _chunk(i):
      cpa = stage(i, 0)
      cpb = stage(i + 1, 1)
      finish(0, *cpa)
      finish(1, *cpb)

    fin = stage(NCHUNK - 1, 0)
    finish(0, *fin)

    plsc.subcore_barrier()
    pltpu.sync_copy(acc_sh.at[pl.ds(s * RPT, RPT)],
                    out_hbm.at[c, pl.ds(s * RPT, RPT)])

    @pl.when(s == NS - 1)
    def _out_tail():
      pltpu.sync_copy(acc_sh.at[pl.ds(RPT * NS, TAIL)],
                      out_hbm.at[c, pl.ds(RPT * NS, TAIL)])

  return _sc_main


def kernel(r, e, a, Wd1, bd1, Wd2, bd2, Wa, W1, b1, W2, b2):
    del Wd1, bd1
    a = a.astype(jnp.int32)
    dbits = lax.bitcast_convert_type(e[:, 0], jnp.int32)
    idx3 = jnp.stack([a[:, 1].reshape(-1, CH), a[:, 0].reshape(-1, CH),
                      dbits.reshape(-1, CH)], axis=1)
    wd2p = jnp.zeros((64, F), jnp.float32).at[:G].set(Wd2)
    rf, tab = _tc_pre(r, Wa, wd2p, bd2.reshape(1, F))
    zeros = jnp.zeros((NA, F), jnp.float32)
    part = _build_sc_main()(idx3, tab, rf, zeros)
    return _tc_post(part[0], part[1], W1, b1.reshape(1, F),
                    W2, b2.reshape(1, F))
